# Initial kernel scaffold; baseline (speedup 1.0000x reference)
import jax
import jax.numpy as jnp
from jax.experimental import pallas as pl


def kernel(x_patient, x_medication, x_disease, x_procedure, x_lab, W1_l, W1_r, b1, W2_l, W2_r, b2, ei_pm, ei_mp, ei_pd, ei_dp, ei_ppr, ei_prp, ei_pl, ei_lp):
    raise NotImplementedError("write your pallas kernel here")


# speedup vs baseline: 4.9714x; 4.9714x over previous
import jax
import jax.numpy as jnp
from jax import lax
from jax.experimental import pallas as pl
from jax.experimental.pallas import tpu as pltpu
from jax.experimental.pallas import tpu_sc as plsc

_NC = 2
_NS = 16
_CH = 128


def _sc_agg(y, src_rows, dst_rows, zeros_w, zeros_c, ones_c, with_cnt):
    NEN, W = y.shape
    ROWS = src_rows.shape[0] // 8
    NE = 8
    Nn = NEN // NE
    rpt = ROWS // _NS
    rem = ROWS % _NS
    npt = (Nn // 8 // _NS) * 8
    nrem = (Nn // 8) % _NS
    ept = NE // _NC

    scratch = [
        pltpu.VMEM((1, _CH), jnp.int32),
        pltpu.VMEM((1, _CH), jnp.int32),
        pltpu.VMEM((_CH, W), jnp.float32),
        pltpu.VMEM_SHARED((Nn, W), jnp.float32),
    ]
    outs = [jax.ShapeDtypeStruct((NEN, W), jnp.float32)]
    if with_cnt:
        scratch += [
            pltpu.VMEM((_CH, 8), jnp.float32),
            pltpu.VMEM_SHARED((Nn, 8), jnp.float32),
        ]
        outs.append(jax.ShapeDtypeStruct((NEN, 8), jnp.float32))
    scratch.append(pltpu.SemaphoreType.DMA)

    mesh = plsc.VectorSubcoreMesh(core_axis_name="c", subcore_axis_name="s")

    def body(*refs):
        if with_cnt:
            (y_hbm, srcr, dstr, z_w, z_c, ones_hbm, agg_hbm, cnt_hbm,
             idx_s, idx_d, rows_v, acc_sh, ones_v, cnt_sh, sem) = refs
        else:
            (y_hbm, srcr, dstr, z_w, agg_hbm,
             idx_s, idx_d, rows_v, acc_sh, sem) = refs
        c = lax.axis_index("c")
        s = lax.axis_index("s")
        if with_cnt:
            pltpu.sync_copy(ones_hbm, ones_v)
        rb = s * npt
        xb = _NS * npt + s * 8

        def tiled_copy(src_ref, dst_ref, so, do):
            pltpu.sync_copy(src_ref.at[pl.ds(so + rb, npt)],
                            dst_ref.at[pl.ds(do + rb, npt)])
            if nrem:
                @pl.when(s < nrem)
                def _():
                    pltpu.sync_copy(src_ref.at[pl.ds(so + xb, 8)],
                                    dst_ref.at[pl.ds(do + xb, 8)])

        def chunk(row):
            pltpu.sync_copy(srcr.at[pl.ds(row, 1)], idx_s)
            pltpu.sync_copy(dstr.at[pl.ds(row, 1)], idx_d)
            pltpu.async_copy(y_hbm.at[idx_s.at[0]], rows_v, sem).wait()
            pltpu.sync_copy(rows_v, acc_sh.at[idx_d.at[0]], add=True)
            if with_cnt:
                pltpu.sync_copy(ones_v, cnt_sh.at[idx_d.at[0]], add=True)

        for j in range(ept):
            e = _NC * j + c
            tiled_copy(z_w, acc_sh, 0, 0)
            if with_cnt:
                tiled_copy(z_c, cnt_sh, 0, 0)
            plsc.subcore_barrier()
            row0 = e * ROWS + s * rpt

            def loop_body(g, carry):
                chunk(row0 + g)
                return carry
            lax.fori_loop(0, rpt, loop_body, 0)
            if rem:
                @pl.when(s < rem)
                def _():
                    chunk(e * ROWS + _NS * rpt + s)
            plsc.subcore_barrier()
            tiled_copy(acc_sh, agg_hbm, 0, e * Nn)
            if with_cnt:
                tiled_copy(cnt_sh, cnt_hbm, 0, e * Nn)
            plsc.subcore_barrier()

    fn = pl.kernel(body, out_type=tuple(outs), mesh=mesh,
                   scratch_types=tuple(scratch),
                   compiler_params=pltpu.CompilerParams(
                       use_tc_tiling_on_sc=False))
    if with_cnt:
        return fn(y, src_rows, dst_rows, zeros_w, zeros_c, ones_c)
    return fn(y, src_rows, dst_rows, zeros_w)


def _tc_transform(x_stack, Wl, bn):
    T, Nn, Win = x_stack.shape
    NE, _, Wout = Wl.shape

    def body(x_ref, w_ref, o_ref):
        o_ref[0] = jnp.dot(x_ref[0], w_ref[0],
                           preferred_element_type=jnp.float32)

    return pl.pallas_call(
        body,
        grid=(NE, Nn // bn),
        in_specs=[
            pl.BlockSpec((1, bn, Win),
                         lambda e, n: (jnp.where(e % 2 == 0, 0, (e + 1) // 2),
                                       n, 0)),
            pl.BlockSpec((1, Win, Wout), lambda e, n: (e, 0, 0)),
        ],
        out_specs=pl.BlockSpec((1, bn, Wout), lambda e, n: (e, n, 0)),
        out_shape=jax.ShapeDtypeStruct((NE, Nn, Wout), jnp.float32),
    )(x_stack, Wl)


def _tc_combine(x_stack, Wr_sum, b_sum, agg, cnt, bn, relu):
    T, Nn, Win = x_stack.shape
    Wout = Wr_sum.shape[2]

    def im_agg(k):
        return lambda d, n: (jnp.where(d == 0, 2 * k + 1, 2 * (d - 1)), n, 0)

    def body(x_ref, w_ref, b_ref, a0, a1, a2, a3, c0, c1, c2, c3, o_ref):
        d = pl.program_id(0)
        out = jnp.dot(x_ref[0], w_ref[0],
                      preferred_element_type=jnp.float32) + b_ref[0]
        out = out + a0[0] / jnp.maximum(c0[0][:, 0:1], 1.0)
        extra = (a1[0] / jnp.maximum(c1[0][:, 0:1], 1.0)
                 + a2[0] / jnp.maximum(c2[0][:, 0:1], 1.0)
                 + a3[0] / jnp.maximum(c3[0][:, 0:1], 1.0))
        out = out + jnp.where(d == 0, 1.0, 0.0) * extra
        if relu:
            out = jnp.maximum(out, 0.0)
        o_ref[0] = out

    in_specs = [
        pl.BlockSpec((1, bn, Win), lambda d, n: (d, n, 0)),
        pl.BlockSpec((1, Win, Wout), lambda d, n: (d, 0, 0)),
        pl.BlockSpec((1, 1, Wout), lambda d, n: (d, 0, 0)),
    ]
    in_specs += [pl.BlockSpec((1, bn, Wout), im_agg(k)) for k in range(4)]
    in_specs += [pl.BlockSpec((1, bn, 8), im_agg(k)) for k in range(4)]
    return pl.pallas_call(
        body,
        grid=(T, Nn // bn),
        in_specs=in_specs,
        out_specs=pl.BlockSpec((1, bn, Wout), lambda d, n: (d, n, 0)),
        out_shape=jax.ShapeDtypeStruct((T, Nn, Wout), jnp.float32),
    )(x_stack, Wr_sum, b_sum[:, None, :], agg, agg, agg, agg,
      cnt, cnt, cnt, cnt)


def kernel(x_patient, x_medication, x_disease, x_procedure, x_lab,
           W1_l, W1_r, b1, W2_l, W2_r, b2,
           ei_pm, ei_mp, ei_pd, ei_dp, ei_ppr, ei_prp, ei_pl, ei_lp):
    Nn, Din = x_patient.shape
    E = ei_pm.shape[1]
    Hh = W1_l.shape[2]
    Oo = W2_l.shape[2]
    NE = W1_l.shape[0]
    bn = 2000 if Nn % 2000 == 0 else Nn

    xs = jnp.stack([x_patient, x_medication, x_disease, x_procedure, x_lab])
    eis = [ei_pm, ei_mp, ei_pd, ei_dp, ei_ppr, ei_prp, ei_pl, ei_lp]
    src = jnp.concatenate([ei[0] + e * Nn for e, ei in enumerate(eis)])
    dst = jnp.concatenate([ei[1] for ei in eis])
    src_rows = src.reshape(-1, _CH)
    dst_rows = dst.reshape(-1, _CH)

    zeros_h = jnp.zeros((Nn, Hh), jnp.float32)
    zeros_o = jnp.zeros((Nn, Oo), jnp.float32)
    zeros_c = jnp.zeros((Nn, 8), jnp.float32)
    ones_c = jnp.ones((_CH, 8), jnp.float32)

    def fold(Wr, b):
        wsum = jnp.stack([Wr[1] + Wr[3] + Wr[5] + Wr[7],
                          Wr[0], Wr[2], Wr[4], Wr[6]])
        bsum = jnp.stack([b[1] + b[3] + b[5] + b[7],
                          b[0], b[2], b[4], b[6]])
        return wsum, bsum
    W1r_s, b1_s = fold(W1_r, b1)
    W2r_s, b2_s = fold(W2_r, b2)

    y1 = _tc_transform(xs, W1_l, bn)
    agg1, cnt = _sc_agg(y1.reshape(NE * Nn, Hh), src_rows, dst_rows,
                        zeros_h, zeros_c, ones_c, with_cnt=True)
    h = _tc_combine(xs, W1r_s, b1_s, agg1.reshape(NE, Nn, Hh),
                    cnt.reshape(NE, Nn, 8), bn, relu=True)
    y2 = _tc_transform(h, W2_l, bn)
    (agg2,) = _sc_agg(y2.reshape(NE * Nn, Oo), src_rows, dst_rows,
                      zeros_o, None, None, with_cnt=False)
    out = _tc_combine(h, W2r_s, b2_s, agg2.reshape(NE, Nn, Oo),
                      cnt.reshape(NE, Nn, 8), bn, relu=False)
    return (out[0], out[1], out[2], out[3], out[4])
